# speedup vs baseline: 1.2840x; 1.2840x over previous
import jax
import jax.numpy as jnp
from jax.experimental import pallas as pl
from jax.experimental.pallas import tpu as pltpu

_F = 16
_KT = 50
_P = 13
_EPS = 1e-5
_NS = (_P - 1 + _KT - 1) // _P + 1


def _stage_a_body(x_ref, w_ref, aff_ref, o_ref):
    p1, na = o_ref.shape
    acc = jnp.dot(x_ref[0:p1], w_ref[0], preferred_element_type=jnp.float32)
    for a in range(1, _NS):
        acc = acc + jnp.dot(x_ref[a:a + p1], w_ref[a],
                            preferred_element_type=jnp.float32)
    m = acc[:, 0:na]
    for r in range(1, _P):
        m = jnp.maximum(m, acc[:, r * na:(r + 1) * na])
    y = jnp.maximum(m + aff_ref[0:1], 0.0) * aff_ref[1:2] + aff_ref[2:3]
    o_ref[...] = y.astype(o_ref.dtype)


def _stage_b_body(z_ref, w_ref, aff_ref, o_ref):
    pc, f = o_ref.shape
    c = (z_ref.shape[0] - pc) // (_NS - 1)
    acc = jnp.dot(z_ref[0:pc], w_ref[0], preferred_element_type=jnp.float32)
    for a in range(1, _NS):
        acc = acc + jnp.dot(z_ref[a * c:a * c + pc], w_ref[a],
                            preferred_element_type=jnp.float32)
    m = acc[:, 0:f]
    for r in range(1, _P):
        m = jnp.maximum(m, acc[:, r * f:(r + 1) * f])
    y = jnp.maximum(m + aff_ref[0:1], 0.0) * aff_ref[1:2] + aff_ref[2:3]
    o_ref[...] = y.astype(o_ref.dtype)


def _head_body(z_ref, w_ref, b_ref, o_ref):
    o_ref[...] = (jnp.dot(z_ref[...], w_ref[...],
                          preferred_element_type=jnp.float32) + b_ref[...])


def kernel(x, w1, b1, w2, b2, w3, b3, gamma1, beta1, mean1, var1,
           gamma2, beta2, mean2, var2, w_lin, b_lin):
    B, T, C = x.shape
    F = _F
    T2 = T - (_KT - 1)
    P1 = (T2 - _P) // _P + 1
    T3 = P1 - (_KT - 1)
    P2 = (T3 - _P) // _P + 1
    E = w_lin.shape[0]
    M1 = P1 + _NS - 1
    M2 = P2 + _NS - 1

    s1 = gamma1 * jax.lax.rsqrt(var1 + _EPS)
    t1 = beta1 - mean1 * s1
    s2 = gamma2 * jax.lax.rsqrt(var2 + _EPS)
    t2 = beta2 - mean2 * s2

    a_i = jnp.arange(_NS)[:, None, None]
    s_i = jnp.arange(_P)[None, :, None]
    r_i = jnp.arange(_P)[None, None, :]
    tap = _P * a_i + s_i - r_i
    ok = (tap >= 0) & (tap < _KT)
    tap_c = jnp.clip(tap, 0, _KT - 1)

    w2_band = jnp.where(ok[..., None], w2.T[tap_c], 0.0)
    w_a = jnp.einsum("ci,asrf->asircf", w1, w2_band)
    w_a = w_a.reshape(_NS, _P * C, _P * C * F).astype(jnp.bfloat16)
    bias_a = (b1[:, None] * jnp.sum(w2, axis=1)[None, :]
              + b2[None, :]).reshape(-1)
    aff_a = jnp.stack([bias_a, jnp.tile(s1, C), jnp.tile(t1, C)], axis=0)

    w3_band = jnp.where(ok[..., None, None],
                        jnp.transpose(w3, (2, 1, 0))[tap_c], 0.0)
    w_b = (jnp.transpose(w3_band, (0, 1, 3, 2, 4))
           .reshape(_NS, _P * F, _P * F).astype(jnp.bfloat16))
    aff_b = jnp.stack([b3, s2, t2], axis=0)

    w_l = (w_lin.reshape(E, F, P2 * C).transpose(2, 1, 0)
           .reshape(P2 * C * F, E).astype(jnp.bfloat16))
    b_l = b_lin.reshape(1, E)

    xb = (jnp.pad(x, ((0, 0), (0, M1 * _P - T), (0, 0)))
          .reshape(B, M1, _P * C).astype(jnp.bfloat16))

    out_a = pl.pallas_call(
        _stage_a_body,
        out_shape=jax.ShapeDtypeStruct((B, P1, C * F), jnp.bfloat16),
        grid=(B,),
        in_specs=[
            pl.BlockSpec((None, M1, _P * C), lambda i: (i, 0, 0)),
            pl.BlockSpec(w_a.shape, lambda i: (0, 0, 0)),
            pl.BlockSpec(aff_a.shape, lambda i: (0, 0)),
        ],
        out_specs=pl.BlockSpec((None, P1, C * F), lambda i: (i, 0, 0)),
        compiler_params=pltpu.CompilerParams(
            dimension_semantics=("parallel",)),
    )(xb, w_a, aff_a)

    zb = (out_a[:, :M2 * _P, :]
          .reshape(B, M2, _P, C, F)
          .transpose(0, 1, 3, 2, 4)
          .reshape(B, M2 * C, _P * F))

    z2 = pl.pallas_call(
        _stage_b_body,
        out_shape=jax.ShapeDtypeStruct((B, P2 * C, F), jnp.bfloat16),
        grid=(B,),
        in_specs=[
            pl.BlockSpec((None, M2 * C, _P * F), lambda i: (i, 0, 0)),
            pl.BlockSpec(w_b.shape, lambda i: (0, 0, 0)),
            pl.BlockSpec(aff_b.shape, lambda i: (0, 0)),
        ],
        out_specs=pl.BlockSpec((None, P2 * C, F), lambda i: (i, 0, 0)),
        compiler_params=pltpu.CompilerParams(
            dimension_semantics=("parallel",)),
    )(zb, w_b, aff_b)

    zf = z2.reshape(B, P2 * C * F)

    bb = min(B, 128)
    out = pl.pallas_call(
        _head_body,
        out_shape=jax.ShapeDtypeStruct((B, E), jnp.float32),
        grid=(pl.cdiv(B, bb),),
        in_specs=[
            pl.BlockSpec((bb, P2 * C * F), lambda i: (i, 0)),
            pl.BlockSpec(w_l.shape, lambda i: (0, 0)),
            pl.BlockSpec(b_l.shape, lambda i: (0, 0)),
        ],
        out_specs=pl.BlockSpec((bb, E), lambda i: (i, 0)),
        compiler_params=pltpu.CompilerParams(
            dimension_semantics=("parallel",)),
    )(zf, w_l, b_l)
    return out


# speedup vs baseline: 1.7203x; 1.3398x over previous
import jax
import jax.numpy as jnp
from jax.experimental import pallas as pl
from jax.experimental.pallas import tpu as pltpu

_F = 16
_KT = 50
_P = 13
_EPS = 1e-5
_NS = (_P - 1 + _KT - 1) // _P + 1


def _stage_a_body(x_ref, w_ref, aff_ref, o_ref):
    ga, p1, na = o_ref.shape
    for g in range(ga):
        acc = jnp.dot(x_ref[g, 0:p1], w_ref[0],
                      preferred_element_type=jnp.float32)
        for a in range(1, _NS):
            acc = acc + jnp.dot(x_ref[g, a:a + p1], w_ref[a],
                                preferred_element_type=jnp.float32)
        m = acc[:, 0:na]
        for r in range(1, _P):
            m = jnp.maximum(m, acc[:, r * na:(r + 1) * na])
        y = jnp.maximum(m + aff_ref[0:1], 0.0) * aff_ref[1:2] + aff_ref[2:3]
        o_ref[g] = y.astype(o_ref.dtype)


def _stage_b_body(z_ref, w_ref, aff_ref, o_ref):
    gb, pc, f = o_ref.shape
    c = (z_ref.shape[1] - pc) // (_NS - 1)
    for g in range(gb):
        acc = jnp.dot(z_ref[g, 0:pc], w_ref[0],
                      preferred_element_type=jnp.float32)
        for a in range(1, _NS):
            acc = acc + jnp.dot(z_ref[g, a * c:a * c + pc], w_ref[a],
                                preferred_element_type=jnp.float32)
        m = acc[:, 0:f]
        for r in range(1, _P):
            m = jnp.maximum(m, acc[:, r * f:(r + 1) * f])
        y = jnp.maximum(m + aff_ref[0:1], 0.0) * aff_ref[1:2] + aff_ref[2:3]
        o_ref[g] = y.astype(o_ref.dtype)


def _head_body(z_ref, w_ref, b_ref, o_ref):
    o_ref[...] = (jnp.dot(z_ref[...], w_ref[...],
                          preferred_element_type=jnp.float32) + b_ref[...])


def kernel(x, w1, b1, w2, b2, w3, b3, gamma1, beta1, mean1, var1,
           gamma2, beta2, mean2, var2, w_lin, b_lin):
    B, T, C = x.shape
    F = _F
    T2 = T - (_KT - 1)
    P1 = (T2 - _P) // _P + 1
    T3 = P1 - (_KT - 1)
    P2 = (T3 - _P) // _P + 1
    E = w_lin.shape[0]
    M1 = P1 + _NS - 1
    M2 = P2 + _NS - 1

    s1 = gamma1 * jax.lax.rsqrt(var1 + _EPS)
    t1 = beta1 - mean1 * s1
    s2 = gamma2 * jax.lax.rsqrt(var2 + _EPS)
    t2 = beta2 - mean2 * s2

    a_i = jnp.arange(_NS)[:, None, None]
    s_i = jnp.arange(_P)[None, :, None]
    r_i = jnp.arange(_P)[None, None, :]
    tap = _P * a_i + s_i - r_i
    ok = (tap >= 0) & (tap < _KT)
    tap_c = jnp.clip(tap, 0, _KT - 1)

    w2_band = jnp.where(ok[..., None], w2.T[tap_c], 0.0)
    w_a = jnp.einsum("ci,asrf->asircf", w1, w2_band)
    w_a = w_a.reshape(_NS, _P * C, _P * C * F).astype(jnp.bfloat16)
    bias_a = (b1[:, None] * jnp.sum(w2, axis=1)[None, :]
              + b2[None, :]).reshape(-1)
    aff_a = jnp.stack([bias_a, jnp.tile(s1, C), jnp.tile(t1, C)], axis=0)

    w3_band = jnp.where(ok[..., None, None],
                        jnp.transpose(w3, (2, 1, 0))[tap_c], 0.0)
    w_b = (jnp.transpose(w3_band, (0, 1, 3, 2, 4))
           .reshape(_NS, _P * F, _P * F).astype(jnp.bfloat16))
    aff_b = jnp.stack([b3, s2, t2], axis=0)

    w_l = (w_lin.reshape(E, F, P2 * C).transpose(2, 1, 0)
           .reshape(P2 * C * F, E).astype(jnp.bfloat16))
    b_l = b_lin.reshape(1, E)

    xb = (jnp.pad(x, ((0, 0), (0, M1 * _P - T), (0, 0)))
          .reshape(B, M1, _P * C).astype(jnp.bfloat16))

    ga = 8
    while B % ga:
        ga //= 2
    out_a = pl.pallas_call(
        _stage_a_body,
        out_shape=jax.ShapeDtypeStruct((B, P1, C * F), jnp.bfloat16),
        grid=(B // ga,),
        in_specs=[
            pl.BlockSpec((ga, M1, _P * C), lambda i: (i, 0, 0)),
            pl.BlockSpec(w_a.shape, lambda i: (0, 0, 0)),
            pl.BlockSpec(aff_a.shape, lambda i: (0, 0)),
        ],
        out_specs=pl.BlockSpec((ga, P1, C * F), lambda i: (i, 0, 0)),
        compiler_params=pltpu.CompilerParams(
            dimension_semantics=("parallel",)),
    )(xb, w_a, aff_a)

    zb = (out_a[:, :M2 * _P, :]
          .reshape(B, M2, _P, C, F)
          .transpose(0, 1, 3, 2, 4)
          .reshape(B, M2 * C, _P * F))

    gb = 16
    while B % gb:
        gb //= 2
    z2 = pl.pallas_call(
        _stage_b_body,
        out_shape=jax.ShapeDtypeStruct((B, P2 * C, F), jnp.bfloat16),
        grid=(B // gb,),
        in_specs=[
            pl.BlockSpec((gb, M2 * C, _P * F), lambda i: (i, 0, 0)),
            pl.BlockSpec(w_b.shape, lambda i: (0, 0, 0)),
            pl.BlockSpec(aff_b.shape, lambda i: (0, 0)),
        ],
        out_specs=pl.BlockSpec((gb, P2 * C, F), lambda i: (i, 0, 0)),
        compiler_params=pltpu.CompilerParams(
            dimension_semantics=("parallel",)),
    )(zb, w_b, aff_b)

    zf = z2.reshape(B, P2 * C * F)

    bb = min(B, 128)
    out = pl.pallas_call(
        _head_body,
        out_shape=jax.ShapeDtypeStruct((B, E), jnp.float32),
        grid=(pl.cdiv(B, bb),),
        in_specs=[
            pl.BlockSpec((bb, P2 * C * F), lambda i: (i, 0)),
            pl.BlockSpec(w_l.shape, lambda i: (0, 0)),
            pl.BlockSpec(b_l.shape, lambda i: (0, 0)),
        ],
        out_specs=pl.BlockSpec((bb, E), lambda i: (i, 0)),
        compiler_params=pltpu.CompilerParams(
            dimension_semantics=("parallel",)),
    )(zf, w_l, b_l)
    return out


# speedup vs baseline: 2.2385x; 1.3012x over previous
import jax
import jax.numpy as jnp
from jax.experimental import pallas as pl
from jax.experimental.pallas import tpu as pltpu

_F = 16
_KT = 50
_P = 13
_EPS = 1e-5
_NS = (_P - 1 + _KT - 1) // _P + 1


def _stage_a_body(x_ref, w_ref, aff_ref, o_ref, xw_ref):
    ga, p1, na = o_ref.shape
    kc = x_ref.shape[2]
    c = kc // _P
    rows = xw_ref.shape[0] // ga
    tail = xw_ref.shape[1] - (_NS - 1) * kc
    for g in range(ga):
        r0 = g * rows
        for a in range(_NS - 1):
            xw_ref[r0:r0 + p1, a * kc:(a + 1) * kc] = x_ref[g, a:a + p1, :]
        xw_ref[r0:r0 + p1, (_NS - 1) * kc:] = \
            x_ref[g, _NS - 1:_NS - 1 + p1, 0:tail]
    acc = jnp.dot(xw_ref[...], w_ref[...], preferred_element_type=jnp.float32)
    m = acc[:, 0:na]
    for r in range(1, _P):
        m = jnp.maximum(m, acc[:, r * na:(r + 1) * na])
    y = jnp.maximum(m + aff_ref[0:1], 0.0) * aff_ref[1:2] + aff_ref[2:3]
    y = y.astype(o_ref.dtype)
    for g in range(ga):
        o_ref[g] = y[g * rows:g * rows + p1]


def _stage_b_body(z_ref, w_ref, aff_ref, o_ref, zw_ref):
    gb, pc, f = o_ref.shape
    kf = z_ref.shape[2]
    c = (z_ref.shape[1] - pc) // (_NS - 1)
    tail = zw_ref.shape[1] - (_NS - 1) * kf
    for g in range(gb):
        r0 = g * pc
        for a in range(_NS - 1):
            zw_ref[r0:r0 + pc, a * kf:(a + 1) * kf] = \
                z_ref[g, a * c:a * c + pc, :]
        zw_ref[r0:r0 + pc, (_NS - 1) * kf:] = \
            z_ref[g, (_NS - 1) * c:(_NS - 1) * c + pc, 0:tail]
    acc = jnp.dot(zw_ref[...], w_ref[...], preferred_element_type=jnp.float32)
    m = acc[:, 0:f]
    for r in range(1, _P):
        m = jnp.maximum(m, acc[:, r * f:(r + 1) * f])
    y = jnp.maximum(m + aff_ref[0:1], 0.0) * aff_ref[1:2] + aff_ref[2:3]
    y = y.astype(o_ref.dtype)
    for g in range(gb):
        o_ref[g] = y[g * pc:(g + 1) * pc]


def _head_body(z_ref, w_ref, b_ref, o_ref):
    o_ref[...] = (jnp.dot(z_ref[...], w_ref[...],
                          preferred_element_type=jnp.float32) + b_ref[...])


def kernel(x, w1, b1, w2, b2, w3, b3, gamma1, beta1, mean1, var1,
           gamma2, beta2, mean2, var2, w_lin, b_lin):
    B, T, C = x.shape
    F = _F
    T2 = T - (_KT - 1)
    P1 = (T2 - _P) // _P + 1
    T3 = P1 - (_KT - 1)
    P2 = (T3 - _P) // _P + 1
    E = w_lin.shape[0]
    M1 = P1 + _NS - 1
    M2 = P2 + _NS - 1

    s1 = gamma1 * jax.lax.rsqrt(var1 + _EPS)
    t1 = beta1 - mean1 * s1
    s2 = gamma2 * jax.lax.rsqrt(var2 + _EPS)
    t2 = beta2 - mean2 * s2

    a_i = jnp.arange(_NS)[:, None, None]
    s_i = jnp.arange(_P)[None, :, None]
    r_i = jnp.arange(_P)[None, None, :]
    tap = _P * a_i + s_i - r_i
    ok = (tap >= 0) & (tap < _KT)
    tap_c = jnp.clip(tap, 0, _KT - 1)

    win_tail = (_P - 1 + _KT - 1 + 1) - (_NS - 1) * _P
    ka = ((_NS - 1) * _P + win_tail) * C

    w2_band = jnp.where(ok[..., None], w2.T[tap_c], 0.0)
    w_a = jnp.einsum("ci,asrf->asircf", w1, w2_band)
    w_a = (w_a.reshape(_NS * _P * C, _P * C * F)[:ka]
           .astype(jnp.bfloat16))
    bias_a = (b1[:, None] * jnp.sum(w2, axis=1)[None, :]
              + b2[None, :]).reshape(-1)
    aff_a = jnp.stack([bias_a, jnp.tile(s1, C), jnp.tile(t1, C)], axis=0)

    kb = ((_NS - 1) * _P + win_tail) * F
    w3_band = jnp.where(ok[..., None, None],
                        jnp.transpose(w3, (2, 1, 0))[tap_c], 0.0)
    w_b = (jnp.transpose(w3_band, (0, 1, 3, 2, 4))
           .reshape(_NS * _P * F, _P * F)[:kb].astype(jnp.bfloat16))
    aff_b = jnp.stack([b3, s2, t2], axis=0)

    w_l = (w_lin.reshape(E, F, P2 * C).transpose(2, 1, 0)
           .reshape(P2 * C * F, E).astype(jnp.bfloat16))
    b_l = b_lin.reshape(1, E)

    xb = (jnp.pad(x, ((0, 0), (0, M1 * _P - T), (0, 0)))
          .reshape(B, M1, _P * C).astype(jnp.bfloat16))

    ga = 8
    while B % ga:
        ga //= 2
    p1r = ((P1 + 7) // 8) * 8
    out_a = pl.pallas_call(
        _stage_a_body,
        out_shape=jax.ShapeDtypeStruct((B, P1, C * F), jnp.bfloat16),
        grid=(B // ga,),
        in_specs=[
            pl.BlockSpec((ga, M1, _P * C), lambda i: (i, 0, 0)),
            pl.BlockSpec(w_a.shape, lambda i: (0, 0)),
            pl.BlockSpec(aff_a.shape, lambda i: (0, 0)),
        ],
        out_specs=pl.BlockSpec((ga, P1, C * F), lambda i: (i, 0, 0)),
        scratch_shapes=[pltpu.VMEM((ga * p1r, ka), jnp.bfloat16)],
        compiler_params=pltpu.CompilerParams(
            dimension_semantics=("parallel",)),
    )(xb, w_a, aff_a)

    zb = (out_a[:, :M2 * _P, :]
          .reshape(B, M2, _P, C, F)
          .transpose(0, 1, 3, 2, 4)
          .reshape(B, M2 * C, _P * F))

    gb = 16
    while B % gb:
        gb //= 2
    z2 = pl.pallas_call(
        _stage_b_body,
        out_shape=jax.ShapeDtypeStruct((B, P2 * C, F), jnp.bfloat16),
        grid=(B // gb,),
        in_specs=[
            pl.BlockSpec((gb, M2 * C, _P * F), lambda i: (i, 0, 0)),
            pl.BlockSpec(w_b.shape, lambda i: (0, 0)),
            pl.BlockSpec(aff_b.shape, lambda i: (0, 0)),
        ],
        out_specs=pl.BlockSpec((gb, P2 * C, F), lambda i: (i, 0, 0)),
        scratch_shapes=[pltpu.VMEM((gb * P2 * C, kb), jnp.bfloat16)],
        compiler_params=pltpu.CompilerParams(
            dimension_semantics=("parallel",)),
    )(zb, w_b, aff_b)

    zf = z2.reshape(B, P2 * C * F)

    bb = min(B, 128)
    out = pl.pallas_call(
        _head_body,
        out_shape=jax.ShapeDtypeStruct((B, E), jnp.float32),
        grid=(pl.cdiv(B, bb),),
        in_specs=[
            pl.BlockSpec((bb, P2 * C * F), lambda i: (i, 0)),
            pl.BlockSpec(w_l.shape, lambda i: (0, 0)),
            pl.BlockSpec(b_l.shape, lambda i: (0, 0)),
        ],
        out_specs=pl.BlockSpec((bb, E), lambda i: (i, 0)),
        compiler_params=pltpu.CompilerParams(
            dimension_semantics=("parallel",)),
    )(zf, w_l, b_l)
    return out
